# baseline (speedup 1.0000x reference)
import functools

import jax
import jax.numpy as jnp
from jax.experimental import pallas as pl
from jax.experimental.pallas import tpu as pltpu

_COEFF = 0.01
_BIG_I = 2**30


def _router_body(x_ref, wg_ref, we_ref, aux_ref):
    xf = x_ref[...]
    wg = wg_ref[...]
    T, _ = xf.shape
    E = wg.shape[0]
    logits = jnp.dot(xf, wg.T, preferred_element_type=jnp.float32)
    iota = jax.lax.broadcasted_iota(jnp.int32, logits.shape, 1)
    m1 = jnp.max(logits, axis=-1, keepdims=True)
    a1 = jnp.min(jnp.where(logits == m1, iota, _BIG_I), axis=-1, keepdims=True)
    oh1 = (iota == a1)
    rest = jnp.where(oh1, -jnp.inf, logits)
    m2 = jnp.max(rest, axis=-1, keepdims=True)
    a2 = jnp.min(jnp.where(rest == m2, iota, _BIG_I), axis=-1, keepdims=True)
    oh2 = (iota == a2)
    e2 = jnp.exp(m2 - m1)
    denom = 1.0 + e2
    w1 = 1.0 / denom
    w2 = e2 / denom
    we = jnp.where(oh1, w1, 0.0) + jnp.where(oh2, w2, 0.0)
    we_ref[...] = jnp.concatenate([we, jnp.ones((T, 1), jnp.float32)], axis=1)
    gates = jnp.exp(logits - m1)
    gates = gates / jnp.sum(gates, axis=-1, keepdims=True)
    f = jnp.mean(oh1.astype(jnp.float32), axis=0)
    P = jnp.mean(gates, axis=0)
    aux_ref[0, 0] = _COEFF * E * jnp.sum(f * P)


def _ffn_body(we_ref, x_ref, gw_ref, uw_ref, dw_ref, out_ref):
    e = pl.program_id(0)
    h = pl.program_id(1)

    @pl.when((e == 0) & (h == 0))
    def _():
        out_ref[...] = jnp.zeros_like(out_ref)

    xf = x_ref[...]
    g = jnp.dot(xf, gw_ref[0].T, preferred_element_type=jnp.float32)
    u = jnp.dot(xf, uw_ref[0].T, preferred_element_type=jnp.float32)
    gelu_g = 0.5 * g * (1.0 + jax.lax.erf(g * (2.0 ** -0.5)))
    hpart = (gelu_g * u).astype(jnp.bfloat16)
    y = jnp.dot(hpart, dw_ref[0], preferred_element_type=jnp.float32)
    EW = we_ref.shape[1]
    oh = (jax.lax.broadcasted_iota(jnp.int32, (EW, 1), 0) == e).astype(jnp.float32)
    we_col = jnp.dot(we_ref[...], oh, preferred_element_type=jnp.float32)
    out_ref[...] += we_col * y


def kernel(x, Wg, gate_w, up_w, down_w, sh_gate, sh_up, sh_down):
    B, T, D = x.shape
    E, H, _ = gate_w.shape
    xf = x.reshape(B * T, D)
    TT = B * T

    we, aux = pl.pallas_call(
        _router_body,
        out_shape=(
            jax.ShapeDtypeStruct((TT, E + 1), jnp.float32),
            jax.ShapeDtypeStruct((1, 1), jnp.float32),
        ),
        out_specs=(
            pl.BlockSpec(memory_space=pltpu.VMEM),
            pl.BlockSpec(memory_space=pltpu.SMEM),
        ),
    )(xf, Wg)

    gw_all = jnp.concatenate([gate_w, sh_gate[None]], axis=0).astype(jnp.bfloat16)
    uw_all = jnp.concatenate([up_w, sh_up[None]], axis=0).astype(jnp.bfloat16)
    dw_all = jnp.concatenate([down_w, sh_down[None]], axis=0)
    dw_all = jnp.swapaxes(dw_all, 1, 2).astype(jnp.bfloat16)
    xf16 = xf.astype(jnp.bfloat16)

    HT = 128
    NH = H // HT
    out = pl.pallas_call(
        _ffn_body,
        grid=(E + 1, NH),
        in_specs=[
            pl.BlockSpec((TT, E + 1), lambda e, h: (0, 0)),
            pl.BlockSpec((TT, D), lambda e, h: (0, 0)),
            pl.BlockSpec((1, HT, D), lambda e, h: (e, h, 0)),
            pl.BlockSpec((1, HT, D), lambda e, h: (e, h, 0)),
            pl.BlockSpec((1, HT, D), lambda e, h: (e, h, 0)),
        ],
        out_specs=pl.BlockSpec((TT, D), lambda e, h: (0, 0)),
        out_shape=jax.ShapeDtypeStruct((TT, D), jnp.float32),
        compiler_params=pltpu.CompilerParams(
            dimension_semantics=("arbitrary", "arbitrary"),
        ),
    )(we, xf16, gw_all, uw_all, dw_all)

    return out.reshape(B, T, D), aux.reshape(())


# speedup vs baseline: 1.6462x; 1.6462x over previous
import functools

import jax
import jax.numpy as jnp
from jax.experimental import pallas as pl
from jax.experimental.pallas import tpu as pltpu

_COEFF = 0.01
_BIG_I = 2**30
_BLK = 256
_HT = 256


def _router_body(x_ref, wg_ref, ea_ref, wa_ref, aux_ref):
    xf = x_ref[...]
    wg = wg_ref[...]
    T, _ = xf.shape
    E = wg.shape[0]
    logits = jnp.dot(xf, wg.T, preferred_element_type=jnp.float32)
    iota = jax.lax.broadcasted_iota(jnp.int32, logits.shape, 1)
    m1 = jnp.max(logits, axis=-1, keepdims=True)
    a1 = jnp.min(jnp.where(logits == m1, iota, _BIG_I), axis=-1, keepdims=True)
    oh1 = (iota == a1)
    rest = jnp.where(oh1, -jnp.inf, logits)
    m2 = jnp.max(rest, axis=-1, keepdims=True)
    a2 = jnp.min(jnp.where(rest == m2, iota, _BIG_I), axis=-1, keepdims=True)
    e2 = jnp.exp(m2 - m1)
    denom = 1.0 + e2
    ea_ref[...] = jnp.concatenate([a1, a2], axis=1)
    wa_ref[...] = jnp.concatenate([1.0 / denom, e2 / denom], axis=1)
    gates = jnp.exp(logits - m1)
    gates = gates / jnp.sum(gates, axis=-1, keepdims=True)
    f = jnp.mean(oh1.astype(jnp.float32), axis=0)
    P = jnp.mean(gates, axis=0)
    aux_ref[0, 0] = _COEFF * E * jnp.sum(f * P)


def _gelu(g):
    return 0.5 * g * (1.0 + jax.lax.erf(g * (2.0 ** -0.5)))


def _moe_ffn_body(blk_e_ref, nblk_ref, xg_ref, gw_ref, uw_ref, dw_ref, y_ref):
    b = pl.program_id(0)
    h = pl.program_id(1)

    @pl.when(b < nblk_ref[0])
    def _():
        xblk = xg_ref[...].astype(jnp.bfloat16)
        g = jnp.dot(xblk, gw_ref[0].T, preferred_element_type=jnp.float32)
        u = jnp.dot(xblk, uw_ref[0].T, preferred_element_type=jnp.float32)
        hpart = (_gelu(g) * u).astype(jnp.bfloat16)

        @pl.when(h == 0)
        def _():
            y_ref[...] = jnp.zeros_like(y_ref)

        y_ref[...] += jnp.dot(hpart, dw_ref[0], preferred_element_type=jnp.float32)


def _sh_ffn_body(x_ref, gw_ref, uw_ref, dw_ref, y_ref):
    h = pl.program_id(1)
    xblk = x_ref[...].astype(jnp.bfloat16)
    g = jnp.dot(xblk, gw_ref[...].T, preferred_element_type=jnp.float32)
    u = jnp.dot(xblk, uw_ref[...].T, preferred_element_type=jnp.float32)
    hpart = (_gelu(g) * u).astype(jnp.bfloat16)

    @pl.when(h == 0)
    def _():
        y_ref[...] = jnp.zeros_like(y_ref)

    y_ref[...] += jnp.dot(hpart, dw_ref[...], preferred_element_type=jnp.float32)


def kernel(x, Wg, gate_w, up_w, down_w, sh_gate, sh_up, sh_down):
    B, T, D = x.shape
    E, H, _ = gate_w.shape
    TT = B * T
    NA = 2 * TT
    NBLK = (NA + E * (_BLK - 1)) // _BLK + 1
    P = NBLK * _BLK
    NH = H // _HT
    xf = x.reshape(TT, D)

    ea, wa, aux = pl.pallas_call(
        _router_body,
        out_shape=(
            jax.ShapeDtypeStruct((TT, 2), jnp.int32),
            jax.ShapeDtypeStruct((TT, 2), jnp.float32),
            jax.ShapeDtypeStruct((1, 1), jnp.float32),
        ),
        out_specs=(
            pl.BlockSpec(memory_space=pltpu.VMEM),
            pl.BlockSpec(memory_space=pltpu.VMEM),
            pl.BlockSpec(memory_space=pltpu.SMEM),
        ),
    )(xf, Wg)

    flat_e = ea.reshape(NA)
    counts = jnp.zeros((E,), jnp.int32).at[flat_e].add(1)
    padded = ((counts + (_BLK - 1)) // _BLK) * _BLK
    group_start = jnp.concatenate([jnp.zeros((1,), jnp.int32),
                                   jnp.cumsum(counts)[:-1].astype(jnp.int32)])
    padded_start = jnp.concatenate([jnp.zeros((1,), jnp.int32),
                                    jnp.cumsum(padded)[:-1].astype(jnp.int32)])
    padded_end = padded_start + padded
    nblk = (jnp.sum(padded) // _BLK).astype(jnp.int32)
    order = jnp.argsort(flat_e, stable=True)
    sorted_e = flat_e[order]
    rank = jnp.arange(NA, dtype=jnp.int32) - group_start[sorted_e]
    pos_sorted = padded_start[sorted_e] + rank
    pos = jnp.zeros((NA,), jnp.int32).at[order].set(pos_sorted)
    src = jnp.zeros((P,), jnp.int32).at[pos_sorted].set(
        (order // 2).astype(jnp.int32))
    blk_ids = jnp.arange(NBLK, dtype=jnp.int32) * _BLK
    blk_e = jnp.sum((blk_ids[:, None] >= padded_end[None, :]).astype(jnp.int32),
                    axis=1).astype(jnp.int32)
    blk_e = jnp.minimum(blk_e, E - 1)
    xg = xf[src]

    gw16 = gate_w.astype(jnp.bfloat16)
    uw16 = up_w.astype(jnp.bfloat16)
    dw16 = jnp.swapaxes(down_w, 1, 2).astype(jnp.bfloat16)

    grid_spec = pltpu.PrefetchScalarGridSpec(
        num_scalar_prefetch=2,
        grid=(NBLK, NH),
        in_specs=[
            pl.BlockSpec((_BLK, D), lambda b, h, be, nb: (b, 0)),
            pl.BlockSpec((1, _HT, D), lambda b, h, be, nb: (be[b], h, 0)),
            pl.BlockSpec((1, _HT, D), lambda b, h, be, nb: (be[b], h, 0)),
            pl.BlockSpec((1, _HT, D), lambda b, h, be, nb: (be[b], h, 0)),
        ],
        out_specs=pl.BlockSpec((_BLK, D), lambda b, h, be, nb: (b, 0)),
    )
    y = pl.pallas_call(
        _moe_ffn_body,
        grid_spec=grid_spec,
        out_shape=jax.ShapeDtypeStruct((P, D), jnp.float32),
        compiler_params=pltpu.CompilerParams(
            dimension_semantics=("arbitrary", "arbitrary"),
        ),
    )(blk_e, jnp.full((1,), nblk, jnp.int32), xg, gw16, uw16, dw16)

    shg16 = sh_gate.astype(jnp.bfloat16)
    shu16 = sh_up.astype(jnp.bfloat16)
    shd16 = jnp.swapaxes(sh_down, 0, 1).astype(jnp.bfloat16)
    TB = min(512, TT)
    NTB = TT // TB
    ysh = pl.pallas_call(
        _sh_ffn_body,
        grid=(NTB, NH),
        in_specs=[
            pl.BlockSpec((TB, D), lambda b, h: (b, 0)),
            pl.BlockSpec((_HT, D), lambda b, h: (h, 0)),
            pl.BlockSpec((_HT, D), lambda b, h: (h, 0)),
            pl.BlockSpec((_HT, D), lambda b, h: (h, 0)),
        ],
        out_specs=pl.BlockSpec((TB, D), lambda b, h: (b, 0)),
        out_shape=jax.ShapeDtypeStruct((TT, D), jnp.float32),
        compiler_params=pltpu.CompilerParams(
            dimension_semantics=("arbitrary", "arbitrary"),
        ),
    )(xf, shg16, shu16, shd16)

    pos2 = pos.reshape(TT, 2)
    out = (wa[:, 0:1] * y[pos2[:, 0]] + wa[:, 1:2] * y[pos2[:, 1]] + ysh)

    return out.reshape(B, T, D), aux.reshape(())
